# baseline (speedup 1.0000x reference)
import functools

import jax
import jax.numpy as jnp
from jax import lax
from jax.experimental import pallas as pl
from jax.experimental.pallas import tpu as pltpu
from jax.experimental.pallas import tpu_sc as plsc

IN_TOKENS_K = 20
VOCAB_K = 100000
EMB_K = 32
BATCH_K = 1024
FAN_IN = IN_TOKENS_K * EMB_K

TN = 2048
NT = (VOCAB_K + TN - 1) // TN

_SC_NW = 32
_SC_CHUNK = 128
_B_TOTAL = BATCH_K * IN_TOKENS_K
_B_PER_W = _B_TOTAL // _SC_NW
_NCHUNK = _B_PER_W // _SC_CHUNK


def _sc_gather(table, idx3d):
    mesh = plsc.VectorSubcoreMesh(core_axis_name="c", subcore_axis_name="s")

    @functools.partial(
        pl.kernel,
        mesh=mesh,
        out_type=jax.ShapeDtypeStruct((_B_TOTAL, EMB_K), jnp.float32),
        scratch_types=[
            pltpu.VMEM((_NCHUNK, _SC_CHUNK), jnp.int32),
            pltpu.VMEM((_B_PER_W, EMB_K), jnp.float32),
            pltpu.SemaphoreType.DMA,
        ],
        compiler_params=pltpu.CompilerParams(use_tc_tiling_on_sc=False),
    )
    def k(table_hbm, idx_hbm, out_hbm, idx_v, rows_v, sem):
        wid = lax.axis_index("s") * 2 + lax.axis_index("c")
        pltpu.sync_copy(idx_hbm.at[wid], idx_v)
        for j in range(_NCHUNK):
            pltpu.async_copy(
                table_hbm.at[idx_v.at[j]],
                rows_v.at[pl.ds(j * _SC_CHUNK, _SC_CHUNK)],
                sem,
            ).wait()
        pltpu.sync_copy(rows_v, out_hbm.at[pl.ds(wid * _B_PER_W, _B_PER_W)])

    return k(table, idx3d)


def _mm_exp_body(a_ref, w_ref, b_ref, e_ref, s_ref, at_ref, acc_ref):
    pid = pl.program_id(0)

    @pl.when(pid == 0)
    def _():
        at_ref[...] = jnp.tanh(a_ref[...])
        acc_ref[...] = jnp.zeros_like(acc_ref)

    logits = jnp.dot(at_ref[...], w_ref[...], preferred_element_type=jnp.float32)
    logits = logits + b_ref[...]
    col = pid * TN + lax.broadcasted_iota(jnp.int32, (1, TN), 1)
    e = jnp.exp(logits)
    e = jnp.where(col < VOCAB_K, e, 0.0)
    e_ref[...] = e
    acc_ref[...] += jnp.sum(e, axis=1, keepdims=True)

    @pl.when(pid == NT - 1)
    def _():
        s_ref[...] = acc_ref[...]


def _norm_body(e_ref, s_ref, o_ref):
    o_ref[...] = e_ref[...] * (1.0 / s_ref[...])


def kernel(x, emb_table, W, b):
    flat_idx = x.reshape(-1).astype(jnp.int32)
    idx3d = flat_idx.reshape(_SC_NW, _NCHUNK, _SC_CHUNK)
    rows = _sc_gather(emb_table, idx3d)
    a = rows.reshape(BATCH_K, FAN_IN)
    b2 = b.reshape(1, VOCAB_K)

    e, s = pl.pallas_call(
        _mm_exp_body,
        grid=(NT,),
        in_specs=[
            pl.BlockSpec((BATCH_K, FAN_IN), lambda i: (0, 0)),
            pl.BlockSpec((FAN_IN, TN), lambda i: (0, i)),
            pl.BlockSpec((1, TN), lambda i: (0, i)),
        ],
        out_specs=[
            pl.BlockSpec((BATCH_K, TN), lambda i: (0, i)),
            pl.BlockSpec((BATCH_K, 1), lambda i: (0, 0)),
        ],
        out_shape=[
            jax.ShapeDtypeStruct((BATCH_K, VOCAB_K), jnp.float32),
            jax.ShapeDtypeStruct((BATCH_K, 1), jnp.float32),
        ],
        scratch_shapes=[
            pltpu.VMEM((BATCH_K, FAN_IN), jnp.float32),
            pltpu.VMEM((BATCH_K, 1), jnp.float32),
        ],
        compiler_params=pltpu.CompilerParams(
            dimension_semantics=("arbitrary",),
        ),
    )(a, W, b2)

    out = pl.pallas_call(
        _norm_body,
        grid=(NT,),
        in_specs=[
            pl.BlockSpec((BATCH_K, TN), lambda i: (0, i)),
            pl.BlockSpec((BATCH_K, 1), lambda i: (0, 0)),
        ],
        out_specs=pl.BlockSpec((BATCH_K, TN), lambda i: (0, i)),
        out_shape=jax.ShapeDtypeStruct((BATCH_K, VOCAB_K), jnp.float32),
        input_output_aliases={0: 0},
        compiler_params=pltpu.CompilerParams(
            dimension_semantics=("arbitrary",),
        ),
    )(e, s)
    return out
